# baseline (speedup 1.0000x reference)
import functools

import jax
import jax.numpy as jnp
from jax import lax
from jax.experimental import pallas as pl
from jax.experimental.pallas import tpu as pltpu
from jax.experimental.pallas import tpu_sc as plsc

_PIX = 16384

_NW = 32
_CHUNK = 128
_NBUF = 8


def _lse_block(pred_ref, msk_ref, sum_ref, cnt_ref):
    b = pl.program_id(0)
    j = pl.program_id(1)

    @pl.when(jnp.logical_and(b == 0, j == 0))
    def _init():
        sum_ref[0, 0] = jnp.float32(0.0)
        cnt_ref[0, 0] = jnp.float32(0.0)

    x = pred_ref[:, :]
    m = msk_ref[0, 0, :]

    mx = jnp.max(x, axis=0)
    s = jnp.sum(jnp.exp(x - mx[None, :]), axis=0)
    lse = mx + jnp.log(s)

    sum_ref[0, 0] += jnp.sum(m * lse)
    cnt_ref[0, 0] += jnp.sum(m)


def _tc_lse(pred2, msk3, B, C, nb):
    return pl.pallas_call(
        _lse_block,
        grid=(B, nb),
        in_specs=[
            pl.BlockSpec((C, _PIX), lambda b, j: (b, j)),
            pl.BlockSpec((1, 1, _PIX), lambda b, j, nb=nb: (b * nb + j, 0, 0)),
        ],
        out_specs=[
            pl.BlockSpec(memory_space=pltpu.SMEM),
            pl.BlockSpec(memory_space=pltpu.SMEM),
        ],
        out_shape=[
            jax.ShapeDtypeStruct((1, 1), jnp.float32),
            jax.ShapeDtypeStruct((1, 1), jnp.float32),
        ],
    )(pred2, msk3)


def _sc_gather_sum(pred_flat, tgt, C, H, W):
    B = tgt.shape[0]
    HW = H * W
    npix = B * HW
    per_w = npix // _NW
    rows_w = per_w // W
    n_outer = per_w // (_CHUNK * _NBUF)
    cpr = W // _CHUNK
    chw = C * HW

    mesh = plsc.VectorSubcoreMesh(core_axis_name="c", subcore_axis_name="s")

    @functools.partial(
        pl.kernel,
        mesh=mesh,
        out_type=jax.ShapeDtypeStruct((_NW, 16), jnp.float32),
        scratch_types=[
            pltpu.VMEM((rows_w, W), jnp.int32),
            pltpu.VMEM((_NBUF, _CHUNK), jnp.int32),
            pltpu.VMEM((_NBUF, _CHUNK), jnp.float32),
            pltpu.VMEM((16,), jnp.float32),
            pltpu.SemaphoreType.DMA,
            pltpu.SemaphoreType.DMA,
        ],
    )
    def sc_fn(pred_hbm, tgt_hbm, out_hbm,
              tgt_v, idx_v, rows_v, acc_v, sem_in, sem_g):
        wid = lax.axis_index("s") * 2 + lax.axis_index("c")
        base = wid * per_w
        bidx = base // HW
        base_flat = bidx * chw + (base % HW)
        r0 = pl.multiple_of((base % HW) // W, rows_w)

        pltpu.async_copy(tgt_hbm.at[bidx, pl.ds(r0, rows_w)], tgt_v,
                         sem_in).wait()
        acc_v[...] = jnp.zeros((16,), jnp.float32)
        lane = lax.iota(jnp.int32, 16)

        def outer(o, _):
            obase = o * (_CHUNK * _NBUF)
            row0 = obase // W
            for j in range(_NBUF):
                row = row0 + j // cpr
                col = (j % cpr) * _CHUNK
                h_abs = r0 + row
                sbase = (bidx * C * HW + (h_abs // 8) * (8 * W)
                         + (j % cpr) * 1024 + (h_abs % 8) * 128)
                for k in range(_CHUNK // 16):
                    t16 = tgt_v[row, pl.ds(col + k * 16, 16)]
                    t16 = jnp.minimum(t16, C - 1)
                    idx_v[j, pl.ds(k * 16, 16)] = (
                        t16 * HW + (sbase + k * 16) + lane)
            gathers = []
            for j in range(_NBUF):
                gathers.append(
                    pltpu.async_copy(pred_hbm.at[idx_v.at[j]], rows_v.at[j],
                                     sem_g))
            for j in range(_NBUF):
                gathers[j].wait()
            for j in range(_NBUF):
                row = row0 + j // cpr
                col = (j % cpr) * _CHUNK
                for k in range(_CHUNK // 16):
                    r16 = rows_v[j, pl.ds(k * 16, 16)]
                    t16 = tgt_v[row, pl.ds(col + k * 16, 16)]
                    acc_v[...] += jnp.where(t16 < C, r16, 0.0)
            return ()

        lax.fori_loop(0, n_outer, outer, (), unroll=False)
        pltpu.sync_copy(acc_v, out_hbm.at[wid])

    return sc_fn(pred_flat, tgt)


def kernel(pred, target, label_mask):
    B, C, H, W = pred.shape
    HW = H * W
    nb = HW // _PIX

    pred2 = pred.reshape(B * C, HW)
    mskf = label_mask.astype(jnp.float32)

    enc = jnp.where(label_mask, target.astype(jnp.int32), jnp.int32(C))
    pred_t = pred.reshape(B, C, H // 8, 8, W // 128, 128)
    pred_t = pred_t.transpose(0, 1, 2, 4, 3, 5).reshape(-1)
    partials = _sc_gather_sum(pred_t, enc, C, H, W)

    total_lse, count = _tc_lse(pred2, mskf.reshape(B * nb, 1, _PIX), B, C, nb)

    total = total_lse[0, 0] - jnp.sum(partials)
    count = count[0, 0]
    safe = jnp.where(count > 0, count, jnp.float32(1.0))
    return jnp.where(count > 0, total / safe, jnp.float32(0.0))


# speedup vs baseline: 1.0261x; 1.0261x over previous
import functools

import jax
import jax.numpy as jnp
from jax import lax
from jax.experimental import pallas as pl
from jax.experimental.pallas import tpu as pltpu
from jax.experimental.pallas import tpu_sc as plsc

_PIX = 32768

_NW = 32
_CHUNK = 128
_NBUF = 8


def _lse_block(pred_ref, msk_ref, sum_ref, cnt_ref):
    b = pl.program_id(0)
    j = pl.program_id(1)

    @pl.when(jnp.logical_and(b == 0, j == 0))
    def _init():
        sum_ref[0, 0] = jnp.float32(0.0)
        cnt_ref[0, 0] = jnp.float32(0.0)

    x = pred_ref[:, :]
    m = msk_ref[0, 0, :]

    mx = jnp.max(x, axis=0)
    s = jnp.sum(jnp.exp(x - mx[None, :]), axis=0)
    lse = mx + jnp.log(s)

    sum_ref[0, 0] += jnp.sum(m * lse)
    cnt_ref[0, 0] += jnp.sum(m)


def _tc_lse(pred2, msk3, B, C, nb):
    return pl.pallas_call(
        _lse_block,
        grid=(B, nb),
        in_specs=[
            pl.BlockSpec((C, _PIX), lambda b, j: (b, j)),
            pl.BlockSpec((1, 1, _PIX), lambda b, j, nb=nb: (b * nb + j, 0, 0)),
        ],
        out_specs=[
            pl.BlockSpec(memory_space=pltpu.SMEM),
            pl.BlockSpec(memory_space=pltpu.SMEM),
        ],
        out_shape=[
            jax.ShapeDtypeStruct((1, 1), jnp.float32),
            jax.ShapeDtypeStruct((1, 1), jnp.float32),
        ],
    )(pred2, msk3)


def _sc_gather_sum(pred_flat, tgt, C, H, W):
    B = tgt.shape[0]
    HW = H * W
    npix = B * HW
    per_w = npix // _NW
    rows_w = per_w // W
    n_outer = per_w // (_CHUNK * _NBUF)
    cpr = W // _CHUNK

    mesh = plsc.VectorSubcoreMesh(core_axis_name="c", subcore_axis_name="s")

    @functools.partial(
        pl.kernel,
        mesh=mesh,
        out_type=jax.ShapeDtypeStruct((_NW, 16), jnp.float32),
        scratch_types=[
            pltpu.VMEM((rows_w, W), jnp.int32),
            pltpu.VMEM((_NBUF, _CHUNK), jnp.int32),
            pltpu.VMEM((_NBUF, _CHUNK), jnp.float32),
            pltpu.VMEM((16,), jnp.float32),
            pltpu.SemaphoreType.DMA,
            pltpu.SemaphoreType.DMA,
        ],
    )
    def sc_fn(pred_hbm, tgt_hbm, out_hbm,
              tgt_v, idx_v, rows_v, acc_v, sem_in, sem_g):
        wid = lax.axis_index("s") * 2 + lax.axis_index("c")
        base = wid * per_w
        bidx = base // HW
        r0 = pl.multiple_of((base % HW) // W, rows_w)

        pltpu.async_copy(tgt_hbm.at[bidx, pl.ds(r0, rows_w)], tgt_v,
                         sem_in).wait()
        acc_v[...] = jnp.zeros((16,), jnp.float32)
        lane = lax.iota(jnp.int32, 16)

        def outer(o, _):
            obase = o * (_CHUNK * _NBUF)
            row0 = obase // W
            for j in range(_NBUF):
                row = row0 + j // cpr
                col = (j % cpr) * _CHUNK
                h_abs = r0 + row
                sbase = (bidx * C * HW + (h_abs // 8) * (8 * W)
                         + (j % cpr) * 1024 + (h_abs % 8) * 128)
                for k in range(_CHUNK // 16):
                    t16 = tgt_v[row, pl.ds(col + k * 16, 16)]
                    t16 = jnp.minimum(t16, C - 1)
                    idx_v[j, pl.ds(k * 16, 16)] = (
                        t16 * HW + (sbase + k * 16) + lane)
            gathers = []
            for j in range(_NBUF):
                gathers.append(
                    pltpu.async_copy(pred_hbm.at[idx_v.at[j]], rows_v.at[j],
                                     sem_g))
            for j in range(_NBUF):
                gathers[j].wait()
            for j in range(_NBUF):
                row = row0 + j // cpr
                col = (j % cpr) * _CHUNK
                for k in range(_CHUNK // 16):
                    r16 = rows_v[j, pl.ds(k * 16, 16)]
                    t16 = tgt_v[row, pl.ds(col + k * 16, 16)]
                    acc_v[...] += jnp.where(t16 < C, r16, 0.0)
            return ()

        lax.fori_loop(0, n_outer, outer, (), unroll=False)
        pltpu.sync_copy(acc_v, out_hbm.at[wid])

    return sc_fn(pred_flat, tgt)


def kernel(pred, target, label_mask):
    B, C, H, W = pred.shape
    HW = H * W
    nb = HW // _PIX

    pred2 = pred.reshape(B * C, HW)
    mskf = label_mask.astype(jnp.float32)

    enc = jnp.where(label_mask, target.astype(jnp.int32), jnp.int32(C))
    pred_t = pred.reshape(B, C, H // 8, 8, W // 128, 128)
    pred_t = pred_t.transpose(0, 1, 2, 4, 3, 5).reshape(-1)
    partials = _sc_gather_sum(pred_t, enc, C, H, W)

    total_lse, count = _tc_lse(pred2, mskf.reshape(B * nb, 1, _PIX), B, C, nb)

    total = total_lse[0, 0] - jnp.sum(partials)
    count = count[0, 0]
    safe = jnp.where(count > 0, count, jnp.float32(1.0))
    return jnp.where(count > 0, total / safe, jnp.float32(0.0))
